# baseline (speedup 1.0000x reference)
import functools

import jax
import jax.numpy as jnp
from jax import lax
from jax.experimental import pallas as pl
from jax.experimental.pallas import tpu as pltpu
from jax.experimental.pallas import tpu_sc as plsc

N = 10000
E = 320000
D = 128
H = 32
L = 64

NC = 2
NS = 16
NW = NC * NS
EW = E // NW
CK = 125
CH = EW // CK
NP = 10240
RPT = NP // NS

BN = 1000


def _sc_mesh():
    return plsc.VectorSubcoreMesh(
        core_axis_name="c", subcore_axis_name="s", num_cores=NC, num_subcores=NS
    )


_SC_PARAMS = pltpu.CompilerParams(use_tc_tiling_on_sc=False)


def _deg_body(dst_hbm, ones_hbm, zeros_hbm, out_hbm, idx_v, ones_v, acc, sem):
    c = lax.axis_index("c")
    s = lax.axis_index("s")
    wid = s * NC + c
    pltpu.sync_copy(zeros_hbm, acc.at[pl.ds(s * RPT, RPT)])
    pltpu.sync_copy(ones_hbm, ones_v)
    pltpu.async_copy(dst_hbm.at[wid], idx_v, sem).wait()
    plsc.subcore_barrier()

    def chunk(ch, carry):
        pltpu.sync_copy(ones_v, acc.at[idx_v.at[ch]], add=True)
        return carry

    lax.fori_loop(0, CH, chunk, 0)
    plsc.subcore_barrier()
    pltpu.sync_copy(acc.at[pl.ds(s * RPT, RPT)], out_hbm.at[c, s])


def _deg_call(dst3, ones_k, zeros1):
    f = pl.kernel(
        _deg_body,
        out_type=jax.ShapeDtypeStruct((NC, NS, RPT), jnp.float32),
        mesh=_sc_mesh(),
        compiler_params=_SC_PARAMS,
        scratch_types=[
            pltpu.VMEM((CH, CK), jnp.int32),
            pltpu.VMEM((CK,), jnp.float32),
            pltpu.VMEM_SHARED((NP,), jnp.float32),
            pltpu.SemaphoreType.DMA,
        ],
    )
    return f(dst3, ones_k, zeros1)


NB = 4
NG = CH // NB


def _agg_body(src_hbm, dst_hbm, h_hbm, zeros_hbm, out_hbm,
              sidx_v, didx_v, rows_v, acc, sem, *sems):
    gsems = sems[:NB]
    ssems = sems[NB:]
    c = lax.axis_index("c")
    s = lax.axis_index("s")
    wid = s * NC + c
    pltpu.sync_copy(zeros_hbm, acc.at[pl.ds(s * RPT, RPT)])
    pltpu.async_copy(src_hbm.at[wid], sidx_v, sem).wait()
    pltpu.async_copy(dst_hbm.at[wid], didx_v, sem).wait()
    plsc.subcore_barrier()

    for b in range(NB):
        pltpu.async_copy(h_hbm.at[sidx_v.at[b]], rows_v.at[b], gsems[b])

    def group(g, carry):
        for b in range(NB):
            pltpu.make_async_copy(
                h_hbm.at[sidx_v.at[0]], rows_v.at[b], gsems[b]).wait()
            pltpu.async_copy(
                rows_v.at[b], acc.at[didx_v.at[g * NB + b]], ssems[b], add=True)

        @pl.when(g < NG - 1)
        def _refill():
            for b in range(NB):
                pltpu.make_async_copy(
                    rows_v.at[b], acc.at[didx_v.at[0]], ssems[b]).wait()
                pltpu.async_copy(
                    h_hbm.at[sidx_v.at[(g + 1) * NB + b]], rows_v.at[b],
                    gsems[b])

        return carry

    lax.fori_loop(0, NG, group, 0)
    for b in range(NB):
        pltpu.make_async_copy(rows_v.at[b], acc.at[didx_v.at[0]], ssems[b]).wait()
    plsc.subcore_barrier()
    pltpu.sync_copy(acc.at[pl.ds(s * RPT, RPT)], out_hbm.at[c, s])


def _agg_call(src3, dst3, h, zerosH):
    f = pl.kernel(
        _agg_body,
        out_type=jax.ShapeDtypeStruct((NC, NS, RPT, H), jnp.float32),
        mesh=_sc_mesh(),
        compiler_params=_SC_PARAMS,
        scratch_types=[
            pltpu.VMEM((CH, CK), jnp.int32),
            pltpu.VMEM((CH, CK), jnp.int32),
            pltpu.VMEM((NB, CK, H), jnp.float32),
            pltpu.VMEM_SHARED((NP, H), jnp.float32),
            pltpu.SemaphoreType.DMA,
        ] + [pltpu.SemaphoreType.DMA] * (2 * NB),
    )
    return f(src3, dst3, h, zerosH)


def _enc1_body(x_ref, w_ref, dinv_ref, o_ref):
    h = jnp.dot(x_ref[...], w_ref[...], preferred_element_type=jnp.float32)
    o_ref[...] = h * dinv_ref[...]


def _enc1(X, W1, dinv):
    return pl.pallas_call(
        _enc1_body,
        grid=(N // BN,),
        in_specs=[
            pl.BlockSpec((BN, D), lambda i: (i, 0)),
            pl.BlockSpec((D, H), lambda i: (0, 0)),
            pl.BlockSpec((BN, 1), lambda i: (i, 0)),
        ],
        out_specs=pl.BlockSpec((BN, H), lambda i: (i, 0)),
        out_shape=jax.ShapeDtypeStruct((N, H), jnp.float32),
    )(X, W1, dinv)


def _enc2_body(p0_ref, p1_ref, hs_ref, dinv_ref, b1_ref, o_ref):
    dinv = dinv_ref[...]
    agg = (p0_ref[...] + p1_ref[...] + hs_ref[...]) * dinv + b1_ref[...]
    o_ref[...] = jnp.maximum(agg, 0.0) * dinv


def _enc2(p0, p1, h1s, dinv, b1):
    bspec = pl.BlockSpec((BN, H), lambda i: (i, 0))
    return pl.pallas_call(
        _enc2_body,
        grid=(N // BN,),
        in_specs=[
            bspec, bspec, bspec,
            pl.BlockSpec((BN, 1), lambda i: (i, 0)),
            pl.BlockSpec((1, H), lambda i: (0, 0)),
        ],
        out_specs=bspec,
        out_shape=jax.ShapeDtypeStruct((N, H), jnp.float32),
    )(p0, p1, h1s, dinv, b1)


def _enc3_body(q0_ref, q1_ref, x1s_ref, dinv_ref, wmu_ref, bmu_ref,
               ws_ref, bs_ref, eps_ref, mus_o, ls_o, z_o):
    p = (q0_ref[...] + q1_ref[...] + x1s_ref[...]) * dinv_ref[...]
    mus = jnp.dot(p, wmu_ref[...], preferred_element_type=jnp.float32) + bmu_ref[...]
    ls = jnp.dot(p, ws_ref[...], preferred_element_type=jnp.float32) + bs_ref[...]
    mus_o[...] = mus
    ls_o[...] = ls
    z_o[...] = mus + jnp.exp(0.5 * ls) * eps_ref[...]


def _enc3(q0, q1, x1s, dinv, Wmu, bmu, Ws, bs, eps):
    bspecH = pl.BlockSpec((BN, H), lambda i: (i, 0))
    return pl.pallas_call(
        _enc3_body,
        grid=(N // BN,),
        in_specs=[
            bspecH, bspecH, bspecH,
            pl.BlockSpec((BN, 1), lambda i: (i, 0)),
            pl.BlockSpec((H, L), lambda i: (0, 0)),
            pl.BlockSpec((1, L), lambda i: (0, 0)),
            pl.BlockSpec((H, 1), lambda i: (0, 0)),
            pl.BlockSpec((1, 1), lambda i: (0, 0)),
            pl.BlockSpec((BN, L), lambda i: (i, 0)),
        ],
        out_specs=[
            pl.BlockSpec((BN, L), lambda i: (i, 0)),
            pl.BlockSpec((BN, 1), lambda i: (i, 0)),
            pl.BlockSpec((BN, L), lambda i: (i, 0)),
        ],
        out_shape=[
            jax.ShapeDtypeStruct((N, L), jnp.float32),
            jax.ShapeDtypeStruct((N, 1), jnp.float32),
            jax.ShapeDtypeStruct((N, L), jnp.float32),
        ],
    )(q0, q1, x1s, dinv, Wmu, bmu, Ws, bs, eps)


def _zzt_body(a_ref, bt_ref, o_ref):
    o_ref[...] = jnp.dot(a_ref[...], bt_ref[...],
                         precision=jax.lax.Precision.HIGHEST,
                         preferred_element_type=jnp.float32)


def _zzt(Z, ZT):
    bm = 1024
    nb = pl.cdiv(N, bm)
    return pl.pallas_call(
        _zzt_body,
        grid=(nb, nb),
        in_specs=[
            pl.BlockSpec((bm, L), lambda i, j: (i, 0)),
            pl.BlockSpec((L, bm), lambda i, j: (0, j)),
        ],
        out_specs=pl.BlockSpec((bm, bm), lambda i, j: (i, j)),
        out_shape=jax.ShapeDtypeStruct((N, N), jnp.float32),
    )(Z, ZT)


def kernel(X, graph, W1, b1, Wmu, bmu, Ws, bs):
    graph = graph.astype(jnp.int32)
    src3 = graph[0].reshape(NW, CH, CK)
    dst3 = graph[1].reshape(NW, CH, CK)
    ones_k = jnp.ones((CK,), jnp.float32)
    zeros1 = jnp.zeros((RPT,), jnp.float32)
    zerosH = jnp.zeros((RPT, H), jnp.float32)

    degp = _deg_call(dst3, ones_k, zeros1).reshape(NC, NP)
    deg = degp[0, :N] + degp[1, :N] + 1.0
    dinv = lax.rsqrt(deg).reshape(N, 1)

    h1s = _enc1(X, W1, dinv)
    p = _agg_call(src3, dst3, h1s, zerosH).reshape(NC, NP, H)[:, :N]
    x1s = _enc2(p[0], p[1], h1s, dinv, b1.reshape(1, H))
    q = _agg_call(src3, dst3, x1s, zerosH).reshape(NC, NP, H)[:, :N]

    eps = jax.random.normal(jax.random.key(1), (N, L), jnp.float32)
    mus, logsigma2s, Z = _enc3(q[0], q[1], x1s, dinv, Wmu,
                               bmu.reshape(1, L), Ws, bs.reshape(1, 1), eps)
    ZZt = _zzt(Z, Z.T)
    return (ZZt, mus, logsigma2s)


# speedup vs baseline: 1.4844x; 1.4844x over previous
import functools

import jax
import jax.numpy as jnp
from jax import lax
from jax.experimental import pallas as pl
from jax.experimental.pallas import tpu as pltpu
from jax.experimental.pallas import tpu_sc as plsc

N = 10000
E = 320000
D = 128
H = 32
L = 64

NC = 2
NS = 16
NW = NC * NS
EW = E // NW
CK = 125
CH = EW // CK
NP = 10240
RPT = NP // NS

BN = 1000


def _sc_mesh():
    return plsc.VectorSubcoreMesh(
        core_axis_name="c", subcore_axis_name="s", num_cores=NC, num_subcores=NS
    )


_SC_PARAMS = pltpu.CompilerParams(use_tc_tiling_on_sc=False)


def _deg_body(dst_hbm, ones_hbm, zeros_hbm, out_hbm, idx_v, ones_v, acc, sem):
    c = lax.axis_index("c")
    s = lax.axis_index("s")
    wid = s * NC + c
    pltpu.sync_copy(zeros_hbm, acc.at[pl.ds(s * RPT, RPT)])
    pltpu.sync_copy(ones_hbm, ones_v)
    pltpu.async_copy(dst_hbm.at[wid], idx_v, sem).wait()
    plsc.subcore_barrier()

    def chunk(ch, carry):
        pltpu.sync_copy(ones_v, acc.at[idx_v.at[ch]], add=True)
        return carry

    lax.fori_loop(0, CH, chunk, 0)
    plsc.subcore_barrier()
    pltpu.sync_copy(acc.at[pl.ds(s * RPT, RPT)], out_hbm.at[c, s])


def _deg_call(dst3, ones_k, zeros1):
    f = pl.kernel(
        _deg_body,
        out_type=jax.ShapeDtypeStruct((NC, NS, RPT), jnp.float32),
        mesh=_sc_mesh(),
        compiler_params=_SC_PARAMS,
        scratch_types=[
            pltpu.VMEM((CH, CK), jnp.int32),
            pltpu.VMEM((CK,), jnp.float32),
            pltpu.VMEM_SHARED((NP,), jnp.float32),
            pltpu.SemaphoreType.DMA,
        ],
    )
    return f(dst3, ones_k, zeros1)


NB = 4
NG = CH // NB


def _agg_body(src_hbm, dst_hbm, h_hbm, zeros_hbm, out_hbm,
              sidx_v, didx_v, rows_v, acc, sem, *sems):
    gsems = sems[:NB]
    ssems = sems[NB:]
    c = lax.axis_index("c")
    s = lax.axis_index("s")
    wid = s * NC + c
    pltpu.sync_copy(zeros_hbm, acc.at[pl.ds(s * RPT, RPT)])
    pltpu.async_copy(src_hbm.at[wid], sidx_v, sem).wait()
    pltpu.async_copy(dst_hbm.at[wid], didx_v, sem).wait()
    plsc.subcore_barrier()

    for b in range(NB):
        pltpu.async_copy(h_hbm.at[sidx_v.at[b]], rows_v.at[b], gsems[b])

    def group(g, carry):
        for b in range(NB):
            pltpu.make_async_copy(
                h_hbm.at[sidx_v.at[0]], rows_v.at[b], gsems[b]).wait()
            pltpu.async_copy(
                rows_v.at[b], acc.at[didx_v.at[g * NB + b]], ssems[b], add=True)

        @pl.when(g < NG - 1)
        def _refill():
            for b in range(NB):
                pltpu.make_async_copy(
                    rows_v.at[b], acc.at[didx_v.at[0]], ssems[b]).wait()
                pltpu.async_copy(
                    h_hbm.at[sidx_v.at[(g + 1) * NB + b]], rows_v.at[b],
                    gsems[b])

        return carry

    lax.fori_loop(0, NG, group, 0)
    for b in range(NB):
        pltpu.make_async_copy(rows_v.at[b], acc.at[didx_v.at[0]], ssems[b]).wait()
    plsc.subcore_barrier()
    pltpu.sync_copy(acc.at[pl.ds(s * RPT, RPT)], out_hbm.at[c, s])


def _agg_call(src3, dst3, h, zerosH):
    f = pl.kernel(
        _agg_body,
        out_type=jax.ShapeDtypeStruct((NC, NS, RPT, H), jnp.float32),
        mesh=_sc_mesh(),
        compiler_params=_SC_PARAMS,
        scratch_types=[
            pltpu.VMEM((CH, CK), jnp.int32),
            pltpu.VMEM((CH, CK), jnp.int32),
            pltpu.VMEM((NB, CK, H), jnp.float32),
            pltpu.VMEM_SHARED((NP, H), jnp.float32),
            pltpu.SemaphoreType.DMA,
        ] + [pltpu.SemaphoreType.DMA] * (2 * NB),
    )
    return f(src3, dst3, h, zerosH)


def _enc1_body(x_ref, w_ref, dinv_ref, o_ref):
    h = jnp.dot(x_ref[...], w_ref[...], preferred_element_type=jnp.float32)
    o_ref[...] = h * dinv_ref[...]


def _enc1(X, W1, dinv):
    return pl.pallas_call(
        _enc1_body,
        grid=(N // BN,),
        in_specs=[
            pl.BlockSpec((BN, D), lambda i: (i, 0)),
            pl.BlockSpec((D, H), lambda i: (0, 0)),
            pl.BlockSpec((BN, 1), lambda i: (i, 0)),
        ],
        out_specs=pl.BlockSpec((BN, H), lambda i: (i, 0)),
        out_shape=jax.ShapeDtypeStruct((N, H), jnp.float32),
    )(X, W1, dinv)


def _enc2_body(p0_ref, p1_ref, hs_ref, dinv_ref, b1_ref, o_ref):
    dinv = dinv_ref[...]
    agg = (p0_ref[...] + p1_ref[...] + hs_ref[...]) * dinv + b1_ref[...]
    o_ref[...] = jnp.maximum(agg, 0.0) * dinv


def _enc2(p0, p1, h1s, dinv, b1):
    bspec = pl.BlockSpec((BN, H), lambda i: (i, 0))
    return pl.pallas_call(
        _enc2_body,
        grid=(N // BN,),
        in_specs=[
            bspec, bspec, bspec,
            pl.BlockSpec((BN, 1), lambda i: (i, 0)),
            pl.BlockSpec((1, H), lambda i: (0, 0)),
        ],
        out_specs=bspec,
        out_shape=jax.ShapeDtypeStruct((N, H), jnp.float32),
    )(p0, p1, h1s, dinv, b1)


def _enc3_body(q0_ref, q1_ref, x1s_ref, dinv_ref, wmu_ref, bmu_ref,
               ws_ref, bs_ref, eps_ref, mus_o, ls_o, z_o):
    p = (q0_ref[...] + q1_ref[...] + x1s_ref[...]) * dinv_ref[...]
    mus = jnp.dot(p, wmu_ref[...], preferred_element_type=jnp.float32) + bmu_ref[...]
    ls = jnp.dot(p, ws_ref[...], preferred_element_type=jnp.float32) + bs_ref[...]
    mus_o[...] = mus
    ls_o[...] = ls
    z_o[...] = mus + jnp.exp(0.5 * ls) * eps_ref[...]


def _enc3(q0, q1, x1s, dinv, Wmu, bmu, Ws, bs, eps):
    bspecH = pl.BlockSpec((BN, H), lambda i: (i, 0))
    return pl.pallas_call(
        _enc3_body,
        grid=(N // BN,),
        in_specs=[
            bspecH, bspecH, bspecH,
            pl.BlockSpec((BN, 1), lambda i: (i, 0)),
            pl.BlockSpec((H, L), lambda i: (0, 0)),
            pl.BlockSpec((1, L), lambda i: (0, 0)),
            pl.BlockSpec((H, 1), lambda i: (0, 0)),
            pl.BlockSpec((1, 1), lambda i: (0, 0)),
            pl.BlockSpec((BN, L), lambda i: (i, 0)),
        ],
        out_specs=[
            pl.BlockSpec((BN, L), lambda i: (i, 0)),
            pl.BlockSpec((BN, 1), lambda i: (i, 0)),
            pl.BlockSpec((BN, L), lambda i: (i, 0)),
        ],
        out_shape=[
            jax.ShapeDtypeStruct((N, L), jnp.float32),
            jax.ShapeDtypeStruct((N, 1), jnp.float32),
            jax.ShapeDtypeStruct((N, L), jnp.float32),
        ],
    )(q0, q1, x1s, dinv, Wmu, bmu, Ws, bs, eps)


def _zzt_body(a_ref, bt_ref, o_ref):
    o_ref[...] = jnp.dot(a_ref[...], bt_ref[...],
                         preferred_element_type=jnp.float32)


def _zzt(Z, ZT):
    bm = 1024
    nb = pl.cdiv(N, bm)
    return pl.pallas_call(
        _zzt_body,
        grid=(nb, nb),
        in_specs=[
            pl.BlockSpec((bm, L), lambda i, j: (i, 0)),
            pl.BlockSpec((L, bm), lambda i, j: (0, j)),
        ],
        out_specs=pl.BlockSpec((bm, bm), lambda i, j: (i, j)),
        out_shape=jax.ShapeDtypeStruct((N, N), jnp.float32),
    )(Z, ZT)


def kernel(X, graph, W1, b1, Wmu, bmu, Ws, bs):
    graph = graph.astype(jnp.int32)
    src3 = graph[0].reshape(NW, CH, CK)
    dst3 = graph[1].reshape(NW, CH, CK)
    ones_k = jnp.ones((CK,), jnp.float32)
    zeros1 = jnp.zeros((RPT,), jnp.float32)
    zerosH = jnp.zeros((RPT, H), jnp.float32)

    degp = _deg_call(dst3, ones_k, zeros1).reshape(NC, NP)
    deg = degp[0, :N] + degp[1, :N] + 1.0
    dinv = lax.rsqrt(deg).reshape(N, 1)

    h1s = _enc1(X, W1, dinv)
    p = _agg_call(src3, dst3, h1s, zerosH).reshape(NC, NP, H)[:, :N]
    x1s = _enc2(p[0], p[1], h1s, dinv, b1.reshape(1, H))
    q = _agg_call(src3, dst3, x1s, zerosH).reshape(NC, NP, H)[:, :N]

    eps = jax.random.normal(jax.random.key(1), (N, L), jnp.float32)
    mus, logsigma2s, Z = _enc3(q[0], q[1], x1s, dinv, Wmu,
                               bmu.reshape(1, L), Ws, bs.reshape(1, 1), eps)
    ZZt = _zzt(Z, Z.T)
    return (ZZt, mus, logsigma2s)


# speedup vs baseline: 1.7068x; 1.1498x over previous
import functools

import jax
import jax.numpy as jnp
from jax import lax
from jax.experimental import pallas as pl
from jax.experimental.pallas import tpu as pltpu
from jax.experimental.pallas import tpu_sc as plsc

N = 10000
E = 320000
D = 128
H = 32
L = 64

NC = 2
NS = 16
NW = NC * NS
EW = E // NW
CK = 125
CH = EW // CK
NP = 10240
RPT = NP // NS

BN = 1000


def _sc_mesh():
    return plsc.VectorSubcoreMesh(
        core_axis_name="c", subcore_axis_name="s", num_cores=NC, num_subcores=NS
    )


_SC_PARAMS = pltpu.CompilerParams(use_tc_tiling_on_sc=False)


def _deg_body(dst_hbm, ones_hbm, zeros_hbm, out_hbm, idx_v, ones_v, acc, sem,
              *ssems):
    c = lax.axis_index("c")
    s = lax.axis_index("s")
    wid = s * NC + c
    pltpu.sync_copy(zeros_hbm, acc.at[pl.ds(s * RPT, RPT)])
    pltpu.sync_copy(ones_hbm, ones_v)
    pltpu.async_copy(dst_hbm.at[wid], idx_v, sem).wait()
    plsc.subcore_barrier()

    for b in range(ND):
        pltpu.async_copy(ones_v, acc.at[idx_v.at[b]], ssems[b], add=True)

    def group(g, carry):
        for b in range(ND):
            pltpu.make_async_copy(ones_v, acc.at[idx_v.at[0]], ssems[b]).wait()
            pltpu.async_copy(ones_v, acc.at[idx_v.at[g * ND + b]], ssems[b],
                             add=True)
        return carry

    lax.fori_loop(1, CH // ND, group, 0)
    for b in range(ND):
        pltpu.make_async_copy(ones_v, acc.at[idx_v.at[0]], ssems[b]).wait()
    plsc.subcore_barrier()
    pltpu.sync_copy(acc.at[pl.ds(s * RPT, RPT)], out_hbm.at[c, s])


def _deg_call(dst3, ones_k, zeros1):
    f = pl.kernel(
        _deg_body,
        out_type=jax.ShapeDtypeStruct((NC, NS, RPT), jnp.float32),
        mesh=_sc_mesh(),
        compiler_params=_SC_PARAMS,
        scratch_types=[
            pltpu.VMEM((CH, CK), jnp.int32),
            pltpu.VMEM((CK,), jnp.float32),
            pltpu.VMEM_SHARED((NP,), jnp.float32),
            pltpu.SemaphoreType.DMA,
        ] + [pltpu.SemaphoreType.DMA] * ND,
    )
    return f(dst3, ones_k, zeros1)


NB = 8
NG = CH // NB
ND = 8


def _agg_body(src_hbm, dst_hbm, h_hbm, zeros_hbm, out_hbm,
              sidx_v, didx_v, rows_v, acc, sem, *sems):
    gsems = sems[:NB]
    ssems = sems[NB:]
    c = lax.axis_index("c")
    s = lax.axis_index("s")
    wid = s * NC + c
    pltpu.sync_copy(zeros_hbm, acc.at[pl.ds(s * RPT, RPT)])
    pltpu.async_copy(src_hbm.at[wid], sidx_v, sem).wait()
    pltpu.async_copy(dst_hbm.at[wid], didx_v, sem).wait()
    plsc.subcore_barrier()

    for b in range(NB):
        pltpu.async_copy(h_hbm.at[sidx_v.at[b]], rows_v.at[b], gsems[b])

    def group(g, carry):
        for b in range(NB):
            pltpu.make_async_copy(
                h_hbm.at[sidx_v.at[0]], rows_v.at[b], gsems[b]).wait()
            pltpu.async_copy(
                rows_v.at[b], acc.at[didx_v.at[g * NB + b]], ssems[b], add=True)

        @pl.when(g < NG - 1)
        def _refill():
            for b in range(NB):
                pltpu.make_async_copy(
                    rows_v.at[b], acc.at[didx_v.at[0]], ssems[b]).wait()
                pltpu.async_copy(
                    h_hbm.at[sidx_v.at[(g + 1) * NB + b]], rows_v.at[b],
                    gsems[b])

        return carry

    lax.fori_loop(0, NG, group, 0)
    for b in range(NB):
        pltpu.make_async_copy(rows_v.at[b], acc.at[didx_v.at[0]], ssems[b]).wait()
    plsc.subcore_barrier()
    pltpu.sync_copy(acc.at[pl.ds(s * RPT, RPT)], out_hbm.at[c, s])


def _agg_call(src3, dst3, h, zerosH):
    f = pl.kernel(
        _agg_body,
        out_type=jax.ShapeDtypeStruct((NC, NS, RPT, H), jnp.float32),
        mesh=_sc_mesh(),
        compiler_params=_SC_PARAMS,
        scratch_types=[
            pltpu.VMEM((CH, CK), jnp.int32),
            pltpu.VMEM((CH, CK), jnp.int32),
            pltpu.VMEM((NB, CK, H), jnp.float32),
            pltpu.VMEM_SHARED((NP, H), jnp.float32),
            pltpu.SemaphoreType.DMA,
        ] + [pltpu.SemaphoreType.DMA] * (2 * NB),
    )
    return f(src3, dst3, h, zerosH)


def _enc1_body(x_ref, w_ref, dinv_ref, o_ref):
    h = jnp.dot(x_ref[...], w_ref[...], preferred_element_type=jnp.float32)
    o_ref[...] = h * dinv_ref[...]


def _enc1(X, W1, dinv):
    return pl.pallas_call(
        _enc1_body,
        grid=(N // BN,),
        in_specs=[
            pl.BlockSpec((BN, D), lambda i: (i, 0)),
            pl.BlockSpec((D, H), lambda i: (0, 0)),
            pl.BlockSpec((BN, 1), lambda i: (i, 0)),
        ],
        out_specs=pl.BlockSpec((BN, H), lambda i: (i, 0)),
        out_shape=jax.ShapeDtypeStruct((N, H), jnp.float32),
    )(X, W1, dinv)


def _enc2_body(p0_ref, p1_ref, hs_ref, dinv_ref, b1_ref, o_ref):
    dinv = dinv_ref[...]
    agg = (p0_ref[...] + p1_ref[...] + hs_ref[...]) * dinv + b1_ref[...]
    o_ref[...] = jnp.maximum(agg, 0.0) * dinv


def _enc2(p0, p1, h1s, dinv, b1):
    bspec = pl.BlockSpec((BN, H), lambda i: (i, 0))
    return pl.pallas_call(
        _enc2_body,
        grid=(N // BN,),
        in_specs=[
            bspec, bspec, bspec,
            pl.BlockSpec((BN, 1), lambda i: (i, 0)),
            pl.BlockSpec((1, H), lambda i: (0, 0)),
        ],
        out_specs=bspec,
        out_shape=jax.ShapeDtypeStruct((N, H), jnp.float32),
    )(p0, p1, h1s, dinv, b1)


def _enc3_body(q0_ref, q1_ref, x1s_ref, dinv_ref, wmu_ref, bmu_ref,
               ws_ref, bs_ref, eps_ref, mus_o, ls_o, z_o):
    p = (q0_ref[...] + q1_ref[...] + x1s_ref[...]) * dinv_ref[...]
    mus = jnp.dot(p, wmu_ref[...], preferred_element_type=jnp.float32) + bmu_ref[...]
    ls = jnp.dot(p, ws_ref[...], preferred_element_type=jnp.float32) + bs_ref[...]
    mus_o[...] = mus
    ls_o[...] = ls
    z_o[...] = mus + jnp.exp(0.5 * ls) * eps_ref[...]


def _enc3(q0, q1, x1s, dinv, Wmu, bmu, Ws, bs, eps):
    bspecH = pl.BlockSpec((BN, H), lambda i: (i, 0))
    return pl.pallas_call(
        _enc3_body,
        grid=(N // BN,),
        in_specs=[
            bspecH, bspecH, bspecH,
            pl.BlockSpec((BN, 1), lambda i: (i, 0)),
            pl.BlockSpec((H, L), lambda i: (0, 0)),
            pl.BlockSpec((1, L), lambda i: (0, 0)),
            pl.BlockSpec((H, 1), lambda i: (0, 0)),
            pl.BlockSpec((1, 1), lambda i: (0, 0)),
            pl.BlockSpec((BN, L), lambda i: (i, 0)),
        ],
        out_specs=[
            pl.BlockSpec((BN, L), lambda i: (i, 0)),
            pl.BlockSpec((BN, 1), lambda i: (i, 0)),
            pl.BlockSpec((BN, L), lambda i: (i, 0)),
        ],
        out_shape=[
            jax.ShapeDtypeStruct((N, L), jnp.float32),
            jax.ShapeDtypeStruct((N, 1), jnp.float32),
            jax.ShapeDtypeStruct((N, L), jnp.float32),
        ],
    )(q0, q1, x1s, dinv, Wmu, bmu, Ws, bs, eps)


def _zzt_body(a_ref, bt_ref, o_ref):
    o_ref[...] = jnp.dot(a_ref[...], bt_ref[...],
                         preferred_element_type=jnp.float32)


def _zzt(Z, ZT):
    bm = 400
    return pl.pallas_call(
        _zzt_body,
        grid=(N // bm,),
        in_specs=[
            pl.BlockSpec((bm, L), lambda i: (i, 0)),
            pl.BlockSpec((L, N), lambda i: (0, 0)),
        ],
        out_specs=pl.BlockSpec((bm, N), lambda i: (i, 0)),
        out_shape=jax.ShapeDtypeStruct((N, N), jnp.float32),
    )(Z, ZT)


def kernel(X, graph, W1, b1, Wmu, bmu, Ws, bs):
    graph = graph.astype(jnp.int32)
    src3 = graph[0].reshape(NW, CH, CK)
    dst3 = graph[1].reshape(NW, CH, CK)
    ones_k = jnp.ones((CK,), jnp.float32)
    zeros1 = jnp.zeros((RPT,), jnp.float32)
    zerosH = jnp.zeros((RPT, H), jnp.float32)

    degp = _deg_call(dst3, ones_k, zeros1).reshape(NC, NP)
    deg = degp[0, :N] + degp[1, :N] + 1.0
    dinv = lax.rsqrt(deg).reshape(N, 1)

    h1s = _enc1(X, W1, dinv)
    p = _agg_call(src3, dst3, h1s, zerosH).reshape(NC, NP, H)[:, :N]
    x1s = _enc2(p[0], p[1], h1s, dinv, b1.reshape(1, H))
    q = _agg_call(src3, dst3, x1s, zerosH).reshape(NC, NP, H)[:, :N]

    eps = jax.random.normal(jax.random.key(1), (N, L), jnp.float32)
    mus, logsigma2s, Z = _enc3(q[0], q[1], x1s, dinv, Wmu,
                               bmu.reshape(1, L), Ws, bs.reshape(1, 1), eps)
    ZZt = _zzt(Z, Z.T)
    return (ZZt, mus, logsigma2s)


# speedup vs baseline: 1.7454x; 1.0226x over previous
import functools

import jax
import jax.numpy as jnp
from jax import lax
from jax.experimental import pallas as pl
from jax.experimental.pallas import tpu as pltpu
from jax.experimental.pallas import tpu_sc as plsc

N = 10000
E = 320000
D = 128
H = 32
L = 64

NC = 2
NS = 16
NW = NC * NS
EW = E // NW
CK = 125
CH = EW // CK
NP = 10240
RPT = NP // NS

BN = 2000


def _sc_mesh():
    return plsc.VectorSubcoreMesh(
        core_axis_name="c", subcore_axis_name="s", num_cores=NC, num_subcores=NS
    )


_SC_PARAMS = pltpu.CompilerParams(use_tc_tiling_on_sc=False)


def _deg_body(dst_hbm, ones_hbm, zeros_hbm, out_hbm, idx_v, ones_v, acc, sem,
              *ssems):
    c = lax.axis_index("c")
    s = lax.axis_index("s")
    wid = s * NC + c
    pltpu.sync_copy(zeros_hbm, acc.at[pl.ds(s * RPT, RPT)])
    pltpu.sync_copy(ones_hbm, ones_v)
    pltpu.async_copy(dst_hbm.at[wid], idx_v, sem).wait()
    plsc.subcore_barrier()

    for b in range(ND):
        pltpu.async_copy(ones_v, acc.at[idx_v.at[b]], ssems[b], add=True)

    def group(g, carry):
        for b in range(ND):
            pltpu.make_async_copy(ones_v, acc.at[idx_v.at[0]], ssems[b]).wait()
            pltpu.async_copy(ones_v, acc.at[idx_v.at[g * ND + b]], ssems[b],
                             add=True)
        return carry

    lax.fori_loop(1, CH // ND, group, 0)
    for b in range(ND):
        pltpu.make_async_copy(ones_v, acc.at[idx_v.at[0]], ssems[b]).wait()
    plsc.subcore_barrier()
    pltpu.sync_copy(acc.at[pl.ds(s * RPT, RPT)], out_hbm.at[c, s])


def _deg_call(dst3, ones_k, zeros1):
    f = pl.kernel(
        _deg_body,
        out_type=jax.ShapeDtypeStruct((NC, NS, RPT), jnp.float32),
        mesh=_sc_mesh(),
        compiler_params=_SC_PARAMS,
        scratch_types=[
            pltpu.VMEM((CH, CK), jnp.int32),
            pltpu.VMEM((CK,), jnp.float32),
            pltpu.VMEM_SHARED((NP,), jnp.float32),
            pltpu.SemaphoreType.DMA,
        ] + [pltpu.SemaphoreType.DMA] * ND,
    )
    return f(dst3, ones_k, zeros1)


NB = 8
NG = CH // NB
ND = 8


def _agg_body(src_hbm, dst_hbm, h_hbm, zeros_hbm, out_hbm,
              sidx_v, didx_v, rows_v, acc, sem, *sems):
    gsems = sems[:NB]
    ssems = sems[NB:]
    c = lax.axis_index("c")
    s = lax.axis_index("s")
    wid = s * NC + c
    pltpu.sync_copy(zeros_hbm, acc.at[pl.ds(s * RPT, RPT)])
    pltpu.async_copy(src_hbm.at[wid], sidx_v, sem).wait()
    pltpu.async_copy(dst_hbm.at[wid], didx_v, sem).wait()
    plsc.subcore_barrier()

    for b in range(NB):
        pltpu.async_copy(h_hbm.at[sidx_v.at[b]], rows_v.at[b], gsems[b])

    def group(g, carry):
        for b in range(NB):
            pltpu.make_async_copy(
                h_hbm.at[sidx_v.at[0]], rows_v.at[b], gsems[b]).wait()
            pltpu.async_copy(
                rows_v.at[b], acc.at[didx_v.at[g * NB + b]], ssems[b], add=True)

        @pl.when(g < NG - 1)
        def _refill():
            for b in range(NB):
                pltpu.make_async_copy(
                    rows_v.at[b], acc.at[didx_v.at[0]], ssems[b]).wait()
                pltpu.async_copy(
                    h_hbm.at[sidx_v.at[(g + 1) * NB + b]], rows_v.at[b],
                    gsems[b])

        return carry

    lax.fori_loop(0, NG, group, 0)
    for b in range(NB):
        pltpu.make_async_copy(rows_v.at[b], acc.at[didx_v.at[0]], ssems[b]).wait()
    plsc.subcore_barrier()
    pltpu.sync_copy(acc.at[pl.ds(s * RPT, RPT)], out_hbm.at[c, s])


def _agg_call(src3, dst3, h, zerosH):
    f = pl.kernel(
        _agg_body,
        out_type=jax.ShapeDtypeStruct((NC, NS, RPT, H), jnp.float32),
        mesh=_sc_mesh(),
        compiler_params=_SC_PARAMS,
        scratch_types=[
            pltpu.VMEM((CH, CK), jnp.int32),
            pltpu.VMEM((CH, CK), jnp.int32),
            pltpu.VMEM((NB, CK, H), jnp.float32),
            pltpu.VMEM_SHARED((NP, H), jnp.float32),
            pltpu.SemaphoreType.DMA,
        ] + [pltpu.SemaphoreType.DMA] * (2 * NB),
    )
    return f(src3, dst3, h, zerosH)


def _enc1_body(x_ref, w_ref, dinv_ref, o_ref):
    h = jnp.dot(x_ref[...], w_ref[...], preferred_element_type=jnp.float32)
    o_ref[...] = h * dinv_ref[...]


def _enc1(X, W1, dinv):
    return pl.pallas_call(
        _enc1_body,
        grid=(N // BN,),
        in_specs=[
            pl.BlockSpec((BN, D), lambda i: (i, 0)),
            pl.BlockSpec((D, H), lambda i: (0, 0)),
            pl.BlockSpec((BN, H), lambda i: (i, 0)),
        ],
        out_specs=pl.BlockSpec((BN, H), lambda i: (i, 0)),
        out_shape=jax.ShapeDtypeStruct((N, H), jnp.float32),
    )(X, W1, dinv)


def _enc2_body(p0_ref, p1_ref, hs_ref, dinv_ref, b1_ref, o_ref):
    dinv = dinv_ref[...]
    agg = (p0_ref[...] + p1_ref[...] + hs_ref[...]) * dinv + b1_ref[...]
    o_ref[...] = jnp.maximum(agg, 0.0) * dinv


def _enc2(p0, p1, h1s, dinv, b1):
    bspec = pl.BlockSpec((BN, H), lambda i: (i, 0))
    return pl.pallas_call(
        _enc2_body,
        grid=(N // BN,),
        in_specs=[
            bspec, bspec, bspec,
            bspec,
            pl.BlockSpec((1, H), lambda i: (0, 0)),
        ],
        out_specs=bspec,
        out_shape=jax.ShapeDtypeStruct((N, H), jnp.float32),
    )(p0, p1, h1s, dinv, b1)


def _enc3_body(q0_ref, q1_ref, x1s_ref, dinv_ref, wmu_ref, bmu_ref,
               ws_ref, bs_ref, eps_ref, mus_o, ls_o, z_o):
    p = (q0_ref[...] + q1_ref[...] + x1s_ref[...]) * dinv_ref[...]
    mus = jnp.dot(p, wmu_ref[...], preferred_element_type=jnp.float32) + bmu_ref[...]
    ls = jnp.dot(p, ws_ref[...], preferred_element_type=jnp.float32) + bs_ref[...]
    mus_o[...] = mus
    ls_o[...] = ls
    z_o[...] = mus + jnp.exp(0.5 * ls) * eps_ref[...]


def _enc3(q0, q1, x1s, dinv, Wmu, bmu, Ws, bs, eps):
    bspecH = pl.BlockSpec((BN, H), lambda i: (i, 0))
    return pl.pallas_call(
        _enc3_body,
        grid=(N // BN,),
        in_specs=[
            bspecH, bspecH, bspecH,
            bspecH,
            pl.BlockSpec((H, L), lambda i: (0, 0)),
            pl.BlockSpec((1, L), lambda i: (0, 0)),
            pl.BlockSpec((H, 1), lambda i: (0, 0)),
            pl.BlockSpec((1, 1), lambda i: (0, 0)),
            pl.BlockSpec((BN, L), lambda i: (i, 0)),
        ],
        out_specs=[
            pl.BlockSpec((BN, L), lambda i: (i, 0)),
            pl.BlockSpec((BN, 1), lambda i: (i, 0)),
            pl.BlockSpec((BN, L), lambda i: (i, 0)),
        ],
        out_shape=[
            jax.ShapeDtypeStruct((N, L), jnp.float32),
            jax.ShapeDtypeStruct((N, 1), jnp.float32),
            jax.ShapeDtypeStruct((N, L), jnp.float32),
        ],
    )(q0, q1, x1s, dinv, Wmu, bmu, Ws, bs, eps)


def _zzt_body(a_ref, bt_ref, o_ref):
    o_ref[...] = jnp.dot(a_ref[...], bt_ref[...],
                         preferred_element_type=jnp.float32)


def _zzt(Z, ZT):
    bm = 400
    return pl.pallas_call(
        _zzt_body,
        grid=(N // bm,),
        in_specs=[
            pl.BlockSpec((bm, L), lambda i: (i, 0)),
            pl.BlockSpec((L, N), lambda i: (0, 0)),
        ],
        out_specs=pl.BlockSpec((bm, N), lambda i: (i, 0)),
        out_shape=jax.ShapeDtypeStruct((N, N), jnp.float32),
    )(Z, ZT)


def kernel(X, graph, W1, b1, Wmu, bmu, Ws, bs):
    graph = graph.astype(jnp.int32)
    src3 = graph[0].reshape(NW, CH, CK)
    dst3 = graph[1].reshape(NW, CH, CK)
    ones_k = jnp.ones((CK,), jnp.float32)
    zeros1 = jnp.zeros((RPT,), jnp.float32)
    zerosH = jnp.zeros((RPT, H), jnp.float32)

    degp = _deg_call(dst3, ones_k, zeros1).reshape(NC, NP)
    deg = degp[0, :N] + degp[1, :N] + 1.0
    dinvH = jnp.broadcast_to(lax.rsqrt(deg)[:, None], (N, H))

    h1s = _enc1(X, W1, dinvH)
    p = _agg_call(src3, dst3, h1s, zerosH).reshape(NC, NP, H)[:, :N]
    x1s = _enc2(p[0], p[1], h1s, dinvH, b1.reshape(1, H))
    q = _agg_call(src3, dst3, x1s, zerosH).reshape(NC, NP, H)[:, :N]

    eps = jax.random.normal(jax.random.key(1), (N, L), jnp.float32)
    mus, logsigma2s, Z = _enc3(q[0], q[1], x1s, dinvH, Wmu,
                               bmu.reshape(1, L), Ws, bs.reshape(1, 1), eps)
    ZZt = _zzt(Z, Z.T)
    return (ZZt, mus, logsigma2s)


# speedup vs baseline: 1.7509x; 1.0031x over previous
import functools

import jax
import jax.numpy as jnp
from jax import lax
from jax.experimental import pallas as pl
from jax.experimental.pallas import tpu as pltpu
from jax.experimental.pallas import tpu_sc as plsc

N = 10000
E = 320000
D = 128
H = 32
L = 64

NC = 2
NS = 16
NW = NC * NS
EW = E // NW
CK = 125
CH = EW // CK
NP = 10240
RPT = NP // NS

BN = 5000


def _sc_mesh():
    return plsc.VectorSubcoreMesh(
        core_axis_name="c", subcore_axis_name="s", num_cores=NC, num_subcores=NS
    )


_SC_PARAMS = pltpu.CompilerParams(use_tc_tiling_on_sc=False)


def _deg_body(dst_hbm, ones_hbm, zeros_hbm, out_hbm, idx_v, ones_v, acc, sem,
              *ssems):
    c = lax.axis_index("c")
    s = lax.axis_index("s")
    wid = s * NC + c
    pltpu.sync_copy(zeros_hbm, acc.at[pl.ds(s * RPT, RPT)])
    pltpu.sync_copy(ones_hbm, ones_v)
    pltpu.async_copy(dst_hbm.at[wid], idx_v, sem).wait()
    plsc.subcore_barrier()

    for b in range(ND):
        pltpu.async_copy(ones_v, acc.at[idx_v.at[b]], ssems[b], add=True)

    def group(g, carry):
        for b in range(ND):
            pltpu.make_async_copy(ones_v, acc.at[idx_v.at[0]], ssems[b]).wait()
            pltpu.async_copy(ones_v, acc.at[idx_v.at[g * ND + b]], ssems[b],
                             add=True)
        return carry

    lax.fori_loop(1, CH // ND, group, 0)
    for b in range(ND):
        pltpu.make_async_copy(ones_v, acc.at[idx_v.at[0]], ssems[b]).wait()
    plsc.subcore_barrier()
    pltpu.sync_copy(acc.at[pl.ds(s * RPT, RPT)], out_hbm.at[c, s])


def _deg_call(dst3, ones_k, zeros1):
    f = pl.kernel(
        _deg_body,
        out_type=jax.ShapeDtypeStruct((NC, NS, RPT), jnp.float32),
        mesh=_sc_mesh(),
        compiler_params=_SC_PARAMS,
        scratch_types=[
            pltpu.VMEM((CH, CK), jnp.int32),
            pltpu.VMEM((CK,), jnp.float32),
            pltpu.VMEM_SHARED((NP,), jnp.float32),
            pltpu.SemaphoreType.DMA,
        ] + [pltpu.SemaphoreType.DMA] * ND,
    )
    return f(dst3, ones_k, zeros1)


NB = 8
NG = CH // NB
ND = 8


def _agg_body(src_hbm, dst_hbm, h_hbm, zeros_hbm, out_hbm,
              sidx_v, didx_v, rows_v, acc, sem, *sems):
    gsems = sems[:NB]
    ssems = sems[NB:]
    c = lax.axis_index("c")
    s = lax.axis_index("s")
    wid = s * NC + c
    pltpu.sync_copy(zeros_hbm, acc.at[pl.ds(s * RPT, RPT)])
    pltpu.async_copy(src_hbm.at[wid], sidx_v, sem).wait()
    pltpu.async_copy(dst_hbm.at[wid], didx_v, sem).wait()
    plsc.subcore_barrier()

    for b in range(NB):
        pltpu.async_copy(h_hbm.at[sidx_v.at[b]], rows_v.at[b], gsems[b])

    def group(g, carry):
        for b in range(NB):
            pltpu.make_async_copy(
                h_hbm.at[sidx_v.at[0]], rows_v.at[b], gsems[b]).wait()
            pltpu.async_copy(
                rows_v.at[b], acc.at[didx_v.at[g * NB + b]], ssems[b], add=True)

        @pl.when(g < NG - 1)
        def _refill():
            for b in range(NB):
                pltpu.make_async_copy(
                    rows_v.at[b], acc.at[didx_v.at[0]], ssems[b]).wait()
                pltpu.async_copy(
                    h_hbm.at[sidx_v.at[(g + 1) * NB + b]], rows_v.at[b],
                    gsems[b])

        return carry

    lax.fori_loop(0, NG, group, 0)
    for b in range(NB):
        pltpu.make_async_copy(rows_v.at[b], acc.at[didx_v.at[0]], ssems[b]).wait()
    plsc.subcore_barrier()
    pltpu.sync_copy(acc.at[pl.ds(s * RPT, RPT)], out_hbm.at[c, s])


def _agg_call(src3, dst3, h, zerosH):
    f = pl.kernel(
        _agg_body,
        out_type=jax.ShapeDtypeStruct((NC, NS, RPT, H), jnp.float32),
        mesh=_sc_mesh(),
        compiler_params=_SC_PARAMS,
        scratch_types=[
            pltpu.VMEM((CH, CK), jnp.int32),
            pltpu.VMEM((CH, CK), jnp.int32),
            pltpu.VMEM((NB, CK, H), jnp.float32),
            pltpu.VMEM_SHARED((NP, H), jnp.float32),
            pltpu.SemaphoreType.DMA,
        ] + [pltpu.SemaphoreType.DMA] * (2 * NB),
    )
    return f(src3, dst3, h, zerosH)


def _enc1_body(x_ref, w_ref, dinv_ref, o_ref):
    h = jnp.dot(x_ref[...], w_ref[...], preferred_element_type=jnp.float32)
    o_ref[...] = h * dinv_ref[...]


def _enc1(X, W1, dinv):
    return pl.pallas_call(
        _enc1_body,
        grid=(N // BN,),
        in_specs=[
            pl.BlockSpec((BN, D), lambda i: (i, 0)),
            pl.BlockSpec((D, H), lambda i: (0, 0)),
            pl.BlockSpec((BN, H), lambda i: (i, 0)),
        ],
        out_specs=pl.BlockSpec((BN, H), lambda i: (i, 0)),
        out_shape=jax.ShapeDtypeStruct((N, H), jnp.float32),
    )(X, W1, dinv)


def _enc2_body(p0_ref, p1_ref, hs_ref, dinv_ref, b1_ref, o_ref):
    dinv = dinv_ref[...]
    agg = (p0_ref[...] + p1_ref[...] + hs_ref[...]) * dinv + b1_ref[...]
    o_ref[...] = jnp.maximum(agg, 0.0) * dinv


def _enc2(p0, p1, h1s, dinv, b1):
    bspec = pl.BlockSpec((BN, H), lambda i: (i, 0))
    return pl.pallas_call(
        _enc2_body,
        grid=(N // BN,),
        in_specs=[
            bspec, bspec, bspec,
            bspec,
            pl.BlockSpec((1, H), lambda i: (0, 0)),
        ],
        out_specs=bspec,
        out_shape=jax.ShapeDtypeStruct((N, H), jnp.float32),
    )(p0, p1, h1s, dinv, b1)


def _enc3_body(q0_ref, q1_ref, x1s_ref, dinv_ref, wmu_ref, bmu_ref,
               ws_ref, bs_ref, eps_ref, mus_o, ls_o, z_o):
    p = (q0_ref[...] + q1_ref[...] + x1s_ref[...]) * dinv_ref[...]
    mus = jnp.dot(p, wmu_ref[...], preferred_element_type=jnp.float32) + bmu_ref[...]
    ls = jnp.dot(p, ws_ref[...], preferred_element_type=jnp.float32) + bs_ref[...]
    mus_o[...] = mus
    ls_o[...] = ls
    z_o[...] = mus + jnp.exp(0.5 * ls) * eps_ref[...]


def _enc3(q0, q1, x1s, dinv, Wmu, bmu, Ws, bs, eps):
    bspecH = pl.BlockSpec((BN, H), lambda i: (i, 0))
    return pl.pallas_call(
        _enc3_body,
        grid=(N // BN,),
        in_specs=[
            bspecH, bspecH, bspecH,
            bspecH,
            pl.BlockSpec((H, L), lambda i: (0, 0)),
            pl.BlockSpec((1, L), lambda i: (0, 0)),
            pl.BlockSpec((H, 1), lambda i: (0, 0)),
            pl.BlockSpec((1, 1), lambda i: (0, 0)),
            pl.BlockSpec((BN, L), lambda i: (i, 0)),
        ],
        out_specs=[
            pl.BlockSpec((BN, L), lambda i: (i, 0)),
            pl.BlockSpec((BN, 1), lambda i: (i, 0)),
            pl.BlockSpec((BN, L), lambda i: (i, 0)),
        ],
        out_shape=[
            jax.ShapeDtypeStruct((N, L), jnp.float32),
            jax.ShapeDtypeStruct((N, 1), jnp.float32),
            jax.ShapeDtypeStruct((N, L), jnp.float32),
        ],
    )(q0, q1, x1s, dinv, Wmu, bmu, Ws, bs, eps)


def _zzt_body(a_ref, bt_ref, o_ref):
    o_ref[...] = jnp.dot(a_ref[...], bt_ref[...],
                         preferred_element_type=jnp.float32)


def _zzt(Z, ZT):
    bm = 400
    return pl.pallas_call(
        _zzt_body,
        grid=(N // bm,),
        in_specs=[
            pl.BlockSpec((bm, L), lambda i: (i, 0)),
            pl.BlockSpec((L, N), lambda i: (0, 0)),
        ],
        out_specs=pl.BlockSpec((bm, N), lambda i: (i, 0)),
        out_shape=jax.ShapeDtypeStruct((N, N), jnp.float32),
    )(Z, ZT)


def kernel(X, graph, W1, b1, Wmu, bmu, Ws, bs):
    graph = graph.astype(jnp.int32)
    src3 = graph[0].reshape(NW, CH, CK)
    dst3 = graph[1].reshape(NW, CH, CK)
    ones_k = jnp.ones((CK,), jnp.float32)
    zeros1 = jnp.zeros((RPT,), jnp.float32)
    zerosH = jnp.zeros((RPT, H), jnp.float32)

    degp = _deg_call(dst3, ones_k, zeros1).reshape(NC, NP)
    deg = degp[0, :N] + degp[1, :N] + 1.0
    dinvH = jnp.broadcast_to(lax.rsqrt(deg)[:, None], (N, H))

    h1s = _enc1(X, W1, dinvH)
    p = _agg_call(src3, dst3, h1s, zerosH).reshape(NC, NP, H)[:, :N]
    x1s = _enc2(p[0], p[1], h1s, dinvH, b1.reshape(1, H))
    q = _agg_call(src3, dst3, x1s, zerosH).reshape(NC, NP, H)[:, :N]

    eps = jax.random.normal(jax.random.key(1), (N, L), jnp.float32)
    mus, logsigma2s, Z = _enc3(q[0], q[1], x1s, dinvH, Wmu,
                               bmu.reshape(1, L), Ws, bs.reshape(1, 1), eps)
    ZZt = _zzt(Z, Z.T)
    return (ZZt, mus, logsigma2s)


# speedup vs baseline: 1.7533x; 1.0014x over previous
import functools

import jax
import jax.numpy as jnp
from jax import lax
from jax.experimental import pallas as pl
from jax.experimental.pallas import tpu as pltpu
from jax.experimental.pallas import tpu_sc as plsc

N = 10000
E = 320000
D = 128
H = 32
L = 64

NC = 2
NS = 16
NW = NC * NS
EW = E // NW
CK = 125
CH = EW // CK
NP = 10240
RPT = NP // NS

BN = 5000


def _sc_mesh():
    return plsc.VectorSubcoreMesh(
        core_axis_name="c", subcore_axis_name="s", num_cores=NC, num_subcores=NS
    )


_SC_PARAMS = pltpu.CompilerParams(use_tc_tiling_on_sc=False)


def _deg_body(dst_hbm, ones_hbm, zeros_hbm, out_hbm, idx_v, ones_v, acc, sem,
              *ssems):
    c = lax.axis_index("c")
    s = lax.axis_index("s")
    wid = s * NC + c
    pltpu.sync_copy(zeros_hbm, acc.at[pl.ds(s * RPT, RPT)])
    pltpu.sync_copy(ones_hbm, ones_v)
    pltpu.async_copy(dst_hbm.at[wid], idx_v, sem).wait()
    plsc.subcore_barrier()

    for b in range(ND):
        pltpu.async_copy(ones_v, acc.at[idx_v.at[b]], ssems[b], add=True)

    def group(g, carry):
        for b in range(ND):
            pltpu.make_async_copy(ones_v, acc.at[idx_v.at[0]], ssems[b]).wait()
            pltpu.async_copy(ones_v, acc.at[idx_v.at[g * ND + b]], ssems[b],
                             add=True)
        return carry

    lax.fori_loop(1, CH // ND, group, 0)
    for b in range(ND):
        pltpu.make_async_copy(ones_v, acc.at[idx_v.at[0]], ssems[b]).wait()
    plsc.subcore_barrier()
    pltpu.sync_copy(acc.at[pl.ds(s * RPT, RPT)], out_hbm.at[c, s])


def _deg_call(dst3, ones_k, zeros1):
    f = pl.kernel(
        _deg_body,
        out_type=jax.ShapeDtypeStruct((NC, NS, RPT), jnp.float32),
        mesh=_sc_mesh(),
        compiler_params=_SC_PARAMS,
        scratch_types=[
            pltpu.VMEM((CH, CK), jnp.int32),
            pltpu.VMEM((CK,), jnp.float32),
            pltpu.VMEM_SHARED((NP,), jnp.float32),
            pltpu.SemaphoreType.DMA,
        ] + [pltpu.SemaphoreType.DMA] * ND,
    )
    return f(dst3, ones_k, zeros1)


NB = 8
NG = CH // NB
ND = 8


def _agg_body(src_hbm, dst_hbm, h_hbm, zeros_hbm, out_hbm,
              sidx_v, didx_v, rows_v, acc, sem, *sems):
    gsems = sems[:NB]
    ssems = sems[NB:]
    c = lax.axis_index("c")
    s = lax.axis_index("s")
    wid = s * NC + c
    pltpu.sync_copy(zeros_hbm, acc.at[pl.ds(s * RPT, RPT)])
    pltpu.async_copy(src_hbm.at[wid], sidx_v, sem).wait()
    pltpu.async_copy(dst_hbm.at[wid], didx_v, sem).wait()
    plsc.subcore_barrier()

    for b in range(NB):
        pltpu.async_copy(h_hbm.at[sidx_v.at[b]], rows_v.at[b], gsems[b])

    def group(g, carry):
        for b in range(NB):
            pltpu.make_async_copy(
                h_hbm.at[sidx_v.at[0]], rows_v.at[b], gsems[b]).wait()
            pltpu.async_copy(
                rows_v.at[b], acc.at[didx_v.at[g * NB + b]], ssems[b], add=True)

        @pl.when(g < NG - 1)
        def _refill():
            for b in range(NB):
                pltpu.make_async_copy(
                    rows_v.at[b], acc.at[didx_v.at[0]], ssems[b]).wait()
                pltpu.async_copy(
                    h_hbm.at[sidx_v.at[(g + 1) * NB + b]], rows_v.at[b],
                    gsems[b])

        return carry

    lax.fori_loop(0, NG, group, 0, unroll=2)
    for b in range(NB):
        pltpu.make_async_copy(rows_v.at[b], acc.at[didx_v.at[0]], ssems[b]).wait()
    plsc.subcore_barrier()
    pltpu.sync_copy(acc.at[pl.ds(s * RPT, RPT)], out_hbm.at[c, s])


def _agg_call(src3, dst3, h, zerosH):
    f = pl.kernel(
        _agg_body,
        out_type=jax.ShapeDtypeStruct((NC, NS, RPT, H), jnp.float32),
        mesh=_sc_mesh(),
        compiler_params=_SC_PARAMS,
        scratch_types=[
            pltpu.VMEM((CH, CK), jnp.int32),
            pltpu.VMEM((CH, CK), jnp.int32),
            pltpu.VMEM((NB, CK, H), jnp.float32),
            pltpu.VMEM_SHARED((NP, H), jnp.float32),
            pltpu.SemaphoreType.DMA,
        ] + [pltpu.SemaphoreType.DMA] * (2 * NB),
    )
    return f(src3, dst3, h, zerosH)


def _enc1_body(x_ref, w_ref, dinv_ref, o_ref):
    h = jnp.dot(x_ref[...], w_ref[...], preferred_element_type=jnp.float32)
    o_ref[...] = h * dinv_ref[...]


def _enc1(X, W1, dinv):
    return pl.pallas_call(
        _enc1_body,
        grid=(N // BN,),
        in_specs=[
            pl.BlockSpec((BN, D), lambda i: (i, 0)),
            pl.BlockSpec((D, H), lambda i: (0, 0)),
            pl.BlockSpec((BN, H), lambda i: (i, 0)),
        ],
        out_specs=pl.BlockSpec((BN, H), lambda i: (i, 0)),
        out_shape=jax.ShapeDtypeStruct((N, H), jnp.float32),
    )(X, W1, dinv)


def _enc2_body(p0_ref, p1_ref, hs_ref, dinv_ref, b1_ref, o_ref):
    dinv = dinv_ref[...]
    agg = (p0_ref[...] + p1_ref[...] + hs_ref[...]) * dinv + b1_ref[...]
    o_ref[...] = jnp.maximum(agg, 0.0) * dinv


def _enc2(p0, p1, h1s, dinv, b1):
    bspec = pl.BlockSpec((BN, H), lambda i: (i, 0))
    return pl.pallas_call(
        _enc2_body,
        grid=(N // BN,),
        in_specs=[
            bspec, bspec, bspec,
            bspec,
            pl.BlockSpec((1, H), lambda i: (0, 0)),
        ],
        out_specs=bspec,
        out_shape=jax.ShapeDtypeStruct((N, H), jnp.float32),
    )(p0, p1, h1s, dinv, b1)


def _enc3_body(q0_ref, q1_ref, x1s_ref, dinv_ref, wmu_ref, bmu_ref,
               ws_ref, bs_ref, eps_ref, mus_o, ls_o, z_o):
    p = (q0_ref[...] + q1_ref[...] + x1s_ref[...]) * dinv_ref[...]
    mus = jnp.dot(p, wmu_ref[...], preferred_element_type=jnp.float32) + bmu_ref[...]
    ls = jnp.dot(p, ws_ref[...], preferred_element_type=jnp.float32) + bs_ref[...]
    mus_o[...] = mus
    ls_o[...] = ls
    z_o[...] = mus + jnp.exp(0.5 * ls) * eps_ref[...]


def _enc3(q0, q1, x1s, dinv, Wmu, bmu, Ws, bs, eps):
    bspecH = pl.BlockSpec((BN, H), lambda i: (i, 0))
    return pl.pallas_call(
        _enc3_body,
        grid=(N // BN,),
        in_specs=[
            bspecH, bspecH, bspecH,
            bspecH,
            pl.BlockSpec((H, L), lambda i: (0, 0)),
            pl.BlockSpec((1, L), lambda i: (0, 0)),
            pl.BlockSpec((H, 1), lambda i: (0, 0)),
            pl.BlockSpec((1, 1), lambda i: (0, 0)),
            pl.BlockSpec((BN, L), lambda i: (i, 0)),
        ],
        out_specs=[
            pl.BlockSpec((BN, L), lambda i: (i, 0)),
            pl.BlockSpec((BN, 1), lambda i: (i, 0)),
            pl.BlockSpec((BN, L), lambda i: (i, 0)),
        ],
        out_shape=[
            jax.ShapeDtypeStruct((N, L), jnp.float32),
            jax.ShapeDtypeStruct((N, 1), jnp.float32),
            jax.ShapeDtypeStruct((N, L), jnp.float32),
        ],
    )(q0, q1, x1s, dinv, Wmu, bmu, Ws, bs, eps)


def _zzt_body(a_ref, bt_ref, o_ref):
    o_ref[...] = jnp.dot(a_ref[...], bt_ref[...],
                         preferred_element_type=jnp.float32)


def _zzt(Z, ZT):
    bm = 400
    return pl.pallas_call(
        _zzt_body,
        grid=(N // bm,),
        in_specs=[
            pl.BlockSpec((bm, L), lambda i: (i, 0)),
            pl.BlockSpec((L, N), lambda i: (0, 0)),
        ],
        out_specs=pl.BlockSpec((bm, N), lambda i: (i, 0)),
        out_shape=jax.ShapeDtypeStruct((N, N), jnp.float32),
    )(Z, ZT)


def kernel(X, graph, W1, b1, Wmu, bmu, Ws, bs):
    graph = graph.astype(jnp.int32)
    src3 = graph[0].reshape(NW, CH, CK)
    dst3 = graph[1].reshape(NW, CH, CK)
    ones_k = jnp.ones((CK,), jnp.float32)
    zeros1 = jnp.zeros((RPT,), jnp.float32)
    zerosH = jnp.zeros((RPT, H), jnp.float32)

    degp = _deg_call(dst3, ones_k, zeros1).reshape(NC, NP)
    deg = degp[0, :N] + degp[1, :N] + 1.0
    dinvH = jnp.broadcast_to(lax.rsqrt(deg)[:, None], (N, H))

    h1s = _enc1(X, W1, dinvH)
    p = _agg_call(src3, dst3, h1s, zerosH).reshape(NC, NP, H)[:, :N]
    x1s = _enc2(p[0], p[1], h1s, dinvH, b1.reshape(1, H))
    q = _agg_call(src3, dst3, x1s, zerosH).reshape(NC, NP, H)[:, :N]

    eps = jax.random.normal(jax.random.key(1), (N, L), jnp.float32)
    mus, logsigma2s, Z = _enc3(q[0], q[1], x1s, dinvH, Wmu,
                               bmu.reshape(1, L), Ws, bs.reshape(1, 1), eps)
    ZZt = _zzt(Z, Z.T)
    return (ZZt, mus, logsigma2s)


# speedup vs baseline: 1.7655x; 1.0070x over previous
import functools

import jax
import jax.numpy as jnp
from jax import lax
from jax.experimental import pallas as pl
from jax.experimental.pallas import tpu as pltpu
from jax.experimental.pallas import tpu_sc as plsc

N = 10000
E = 320000
D = 128
H = 32
L = 64

NC = 2
NS = 16
NW = NC * NS
EW = E // NW
CK = 125
CH = EW // CK
NP = 10240
RPT = NP // NS

BN = 5000


def _sc_mesh():
    return plsc.VectorSubcoreMesh(
        core_axis_name="c", subcore_axis_name="s", num_cores=NC, num_subcores=NS
    )


_SC_PARAMS = pltpu.CompilerParams(use_tc_tiling_on_sc=False)


def _deg_body(dst_hbm, ones_hbm, zeros_hbm, out_hbm, idx_v, ones_v, acc, sem,
              *ssems):
    c = lax.axis_index("c")
    s = lax.axis_index("s")
    wid = s * NC + c
    pltpu.sync_copy(zeros_hbm, acc.at[pl.ds(s * RPT, RPT)])
    pltpu.sync_copy(ones_hbm, ones_v)
    pltpu.async_copy(dst_hbm.at[wid], idx_v, sem).wait()
    plsc.subcore_barrier()

    for b in range(ND):
        pltpu.async_copy(ones_v, acc.at[idx_v.at[b]], ssems[b], add=True)

    def group(g, carry):
        for b in range(ND):
            pltpu.make_async_copy(ones_v, acc.at[idx_v.at[0]], ssems[b]).wait()
            pltpu.async_copy(ones_v, acc.at[idx_v.at[g * ND + b]], ssems[b],
                             add=True)
        return carry

    lax.fori_loop(1, CH // ND, group, 0)
    for b in range(ND):
        pltpu.make_async_copy(ones_v, acc.at[idx_v.at[0]], ssems[b]).wait()
    plsc.subcore_barrier()
    pltpu.sync_copy(acc.at[pl.ds(s * RPT, RPT)], out_hbm.at[c, s])


def _deg_call(dst3, ones_k, zeros1):
    f = pl.kernel(
        _deg_body,
        out_type=jax.ShapeDtypeStruct((NC, NS, RPT), jnp.float32),
        mesh=_sc_mesh(),
        compiler_params=_SC_PARAMS,
        scratch_types=[
            pltpu.VMEM((CH, CK), jnp.int32),
            pltpu.VMEM((CK,), jnp.float32),
            pltpu.VMEM_SHARED((NP,), jnp.float32),
            pltpu.SemaphoreType.DMA,
        ] + [pltpu.SemaphoreType.DMA] * ND,
    )
    return f(dst3, ones_k, zeros1)


NB = 8
NG = CH // NB
ND = 8


def _agg_body(src_hbm, dst_hbm, h_hbm, zeros_hbm, out_hbm,
              sidx_v, didx_v, rows_v, acc, sem, *sems):
    gsems = sems[:NB]
    ssems = sems[NB:]
    c = lax.axis_index("c")
    s = lax.axis_index("s")
    wid = s * NC + c
    pltpu.sync_copy(zeros_hbm, acc.at[pl.ds(s * RPT, RPT)])
    pltpu.async_copy(src_hbm.at[wid], sidx_v, sem).wait()
    pltpu.async_copy(dst_hbm.at[wid], didx_v, sem).wait()
    plsc.subcore_barrier()

    for b in range(NB):
        pltpu.async_copy(h_hbm.at[sidx_v.at[b]], rows_v.at[b], gsems[b])

    def group(g, carry):
        for b in range(NB):
            pltpu.make_async_copy(
                h_hbm.at[sidx_v.at[0]], rows_v.at[b], gsems[b]).wait()
            pltpu.async_copy(
                rows_v.at[b], acc.at[didx_v.at[g * NB + b]], ssems[b], add=True)

        @pl.when(g < NG - 1)
        def _refill():
            for b in range(NB):
                pltpu.make_async_copy(
                    rows_v.at[b], acc.at[didx_v.at[0]], ssems[b]).wait()
                pltpu.async_copy(
                    h_hbm.at[sidx_v.at[(g + 1) * NB + b]], rows_v.at[b],
                    gsems[b])

        return carry

    lax.fori_loop(0, NG, group, 0, unroll=2)
    for b in range(NB):
        pltpu.make_async_copy(rows_v.at[b], acc.at[didx_v.at[0]], ssems[b]).wait()
    plsc.subcore_barrier()
    pltpu.sync_copy(acc.at[pl.ds(s * RPT, RPT)], out_hbm.at[c, s])


def _agg_call(src3, dst3, h, zerosH):
    f = pl.kernel(
        _agg_body,
        out_type=jax.ShapeDtypeStruct((NC, NS, RPT, H), jnp.float32),
        mesh=_sc_mesh(),
        compiler_params=_SC_PARAMS,
        scratch_types=[
            pltpu.VMEM((CH, CK), jnp.int32),
            pltpu.VMEM((CH, CK), jnp.int32),
            pltpu.VMEM((NB, CK, H), jnp.float32),
            pltpu.VMEM_SHARED((NP, H), jnp.float32),
            pltpu.SemaphoreType.DMA,
        ] + [pltpu.SemaphoreType.DMA] * (2 * NB),
    )
    return f(src3, dst3, h, zerosH)


def _enc1_body(x_ref, w_ref, dinv_ref, o_ref):
    h = jnp.dot(x_ref[...], w_ref[...], preferred_element_type=jnp.float32)
    o_ref[...] = h * dinv_ref[...]


def _enc1(X, W1, dinv):
    return pl.pallas_call(
        _enc1_body,
        grid=(N // BN,),
        in_specs=[
            pl.BlockSpec((BN, D), lambda i: (i, 0)),
            pl.BlockSpec((D, H), lambda i: (0, 0)),
            pl.BlockSpec((BN, H), lambda i: (i, 0)),
        ],
        out_specs=pl.BlockSpec((BN, H), lambda i: (i, 0)),
        out_shape=jax.ShapeDtypeStruct((N, H), jnp.float32),
    )(X, W1, dinv)


def _enc2_body(p0_ref, p1_ref, hs_ref, dinv_ref, b1_ref, o_ref):
    dinv = dinv_ref[...]
    agg = (p0_ref[...] + p1_ref[...] + hs_ref[...]) * dinv + b1_ref[...]
    o_ref[...] = jnp.maximum(agg, 0.0) * dinv


def _enc2(p0, p1, h1s, dinv, b1):
    bspec = pl.BlockSpec((BN, H), lambda i: (i, 0))
    return pl.pallas_call(
        _enc2_body,
        grid=(N // BN,),
        in_specs=[
            bspec, bspec, bspec,
            bspec,
            pl.BlockSpec((1, H), lambda i: (0, 0)),
        ],
        out_specs=bspec,
        out_shape=jax.ShapeDtypeStruct((N, H), jnp.float32),
    )(p0, p1, h1s, dinv, b1)


def _enc3_body(q0_ref, q1_ref, x1s_ref, dinv_ref, wmu_ref, bmu_ref,
               ws_ref, bs_ref, eps_ref, mus_o, ls_o, z_o):
    p = (q0_ref[...] + q1_ref[...] + x1s_ref[...]) * dinv_ref[...]
    mus = jnp.dot(p, wmu_ref[...], preferred_element_type=jnp.float32) + bmu_ref[...]
    ls = jnp.dot(p, ws_ref[...], preferred_element_type=jnp.float32) + bs_ref[...]
    mus_o[...] = mus
    ls_o[...] = ls
    z_o[...] = mus + jnp.exp(0.5 * ls) * eps_ref[...]


def _enc3(q0, q1, x1s, dinv, Wmu, bmu, Ws, bs, eps):
    bspecH = pl.BlockSpec((BN, H), lambda i: (i, 0))
    return pl.pallas_call(
        _enc3_body,
        grid=(N // BN,),
        in_specs=[
            bspecH, bspecH, bspecH,
            bspecH,
            pl.BlockSpec((H, L), lambda i: (0, 0)),
            pl.BlockSpec((1, L), lambda i: (0, 0)),
            pl.BlockSpec((H, 1), lambda i: (0, 0)),
            pl.BlockSpec((1, 1), lambda i: (0, 0)),
            pl.BlockSpec((BN, L), lambda i: (i, 0)),
        ],
        out_specs=[
            pl.BlockSpec((BN, L), lambda i: (i, 0)),
            pl.BlockSpec((BN, 1), lambda i: (i, 0)),
            pl.BlockSpec((BN, L), lambda i: (i, 0)),
        ],
        out_shape=[
            jax.ShapeDtypeStruct((N, L), jnp.float32),
            jax.ShapeDtypeStruct((N, 1), jnp.float32),
            jax.ShapeDtypeStruct((N, L), jnp.float32),
        ],
    )(q0, q1, x1s, dinv, Wmu, bmu, Ws, bs, eps)


def _zzt_body(a_ref, bt_ref, o_ref):
    o_ref[...] = jnp.dot(a_ref[...], bt_ref[...],
                         preferred_element_type=jnp.float32)


def _zzt(Z, ZT):
    bm = 200
    return pl.pallas_call(
        _zzt_body,
        grid=(N // bm,),
        in_specs=[
            pl.BlockSpec((bm, L), lambda i: (i, 0)),
            pl.BlockSpec((L, N), lambda i: (0, 0)),
        ],
        out_specs=pl.BlockSpec((bm, N), lambda i: (i, 0)),
        out_shape=jax.ShapeDtypeStruct((N, N), jnp.float32),
    )(Z, ZT)


def kernel(X, graph, W1, b1, Wmu, bmu, Ws, bs):
    graph = graph.astype(jnp.int32)
    src3 = graph[0].reshape(NW, CH, CK)
    dst3 = graph[1].reshape(NW, CH, CK)
    ones_k = jnp.ones((CK,), jnp.float32)
    zeros1 = jnp.zeros((RPT,), jnp.float32)
    zerosH = jnp.zeros((RPT, H), jnp.float32)

    degp = _deg_call(dst3, ones_k, zeros1).reshape(NC, NP)
    deg = degp[0, :N] + degp[1, :N] + 1.0
    dinvH = jnp.broadcast_to(lax.rsqrt(deg)[:, None], (N, H))

    h1s = _enc1(X, W1, dinvH)
    p = _agg_call(src3, dst3, h1s, zerosH).reshape(NC, NP, H)[:, :N]
    x1s = _enc2(p[0], p[1], h1s, dinvH, b1.reshape(1, H))
    q = _agg_call(src3, dst3, x1s, zerosH).reshape(NC, NP, H)[:, :N]

    eps = jax.random.normal(jax.random.key(1), (N, L), jnp.float32)
    mus, logsigma2s, Z = _enc3(q[0], q[1], x1s, dinvH, Wmu,
                               bmu.reshape(1, L), Ws, bs.reshape(1, 1), eps)
    ZZt = _zzt(Z, Z.T)
    return (ZZt, mus, logsigma2s)


# speedup vs baseline: 1.8004x; 1.0197x over previous
import functools

import jax
import jax.numpy as jnp
from jax import lax
from jax.experimental import pallas as pl
from jax.experimental.pallas import tpu as pltpu
from jax.experimental.pallas import tpu_sc as plsc

N = 10000
E = 320000
D = 128
H = 32
L = 64

NC = 2
NS = 16
NW = NC * NS
EW = E // NW
CK = 125
CH = EW // CK
NP = 10240
RPT = NP // NS

BN = 5000


def _sc_mesh():
    return plsc.VectorSubcoreMesh(
        core_axis_name="c", subcore_axis_name="s", num_cores=NC, num_subcores=NS
    )


_SC_PARAMS = pltpu.CompilerParams(use_tc_tiling_on_sc=False)


def _deg_body(g_hbm, ones_hbm, zeros_hbm, out_hbm, idx_v, ones_v, acc, sem,
              *ssems):
    c = lax.axis_index("c")
    s = lax.axis_index("s")
    wid = s * NC + c
    pltpu.sync_copy(zeros_hbm, acc.at[pl.ds(s * RPT, RPT)])
    pltpu.sync_copy(ones_hbm, ones_v)
    pltpu.async_copy(g_hbm.at[1, wid], idx_v, sem).wait()
    plsc.subcore_barrier()

    for b in range(ND):
        pltpu.async_copy(ones_v, acc.at[idx_v.at[b]], ssems[b], add=True)

    def group(g, carry):
        for b in range(ND):
            pltpu.make_async_copy(ones_v, acc.at[idx_v.at[0]], ssems[b]).wait()
            pltpu.async_copy(ones_v, acc.at[idx_v.at[g * ND + b]], ssems[b],
                             add=True)
        return carry

    lax.fori_loop(1, CH // ND, group, 0)
    for b in range(ND):
        pltpu.make_async_copy(ones_v, acc.at[idx_v.at[0]], ssems[b]).wait()
    plsc.subcore_barrier()
    pltpu.sync_copy(acc.at[pl.ds(s * RPT, RPT)], out_hbm.at[c, s])


def _deg_call(g4, ones_k, zeros1):
    f = pl.kernel(
        _deg_body,
        out_type=jax.ShapeDtypeStruct((NC, NS, RPT), jnp.float32),
        mesh=_sc_mesh(),
        compiler_params=_SC_PARAMS,
        scratch_types=[
            pltpu.VMEM((CH, CK), jnp.int32),
            pltpu.VMEM((CK,), jnp.float32),
            pltpu.VMEM_SHARED((NP,), jnp.float32),
            pltpu.SemaphoreType.DMA,
        ] + [pltpu.SemaphoreType.DMA] * ND,
    )
    return f(g4, ones_k, zeros1)


NB = 8
NG = CH // NB
ND = 8


def _agg_body(g_hbm, h_hbm, zeros_hbm, out_hbm,
              sidx_v, didx_v, rows_v, acc, sem, *sems):
    gsems = sems[:NB]
    ssems = sems[NB:]
    c = lax.axis_index("c")
    s = lax.axis_index("s")
    wid = s * NC + c
    pltpu.sync_copy(zeros_hbm, acc.at[pl.ds(s * RPT, RPT)])
    pltpu.async_copy(g_hbm.at[0, wid], sidx_v, sem).wait()
    pltpu.async_copy(g_hbm.at[1, wid], didx_v, sem).wait()
    plsc.subcore_barrier()

    for b in range(NB):
        pltpu.async_copy(h_hbm.at[sidx_v.at[b]], rows_v.at[b], gsems[b])

    def group(g, carry):
        for b in range(NB):
            pltpu.make_async_copy(
                h_hbm.at[sidx_v.at[0]], rows_v.at[b], gsems[b]).wait()
            pltpu.async_copy(
                rows_v.at[b], acc.at[didx_v.at[g * NB + b]], ssems[b], add=True)

        @pl.when(g < NG - 1)
        def _refill():
            for b in range(NB):
                pltpu.make_async_copy(
                    rows_v.at[b], acc.at[didx_v.at[0]], ssems[b]).wait()
                pltpu.async_copy(
                    h_hbm.at[sidx_v.at[(g + 1) * NB + b]], rows_v.at[b],
                    gsems[b])

        return carry

    lax.fori_loop(0, NG, group, 0, unroll=2)
    for b in range(NB):
        pltpu.make_async_copy(rows_v.at[b], acc.at[didx_v.at[0]], ssems[b]).wait()
    plsc.subcore_barrier()
    pltpu.sync_copy(acc.at[pl.ds(s * RPT, RPT)], out_hbm.at[c, s])


def _agg_call(g4, h, zerosH):
    f = pl.kernel(
        _agg_body,
        out_type=jax.ShapeDtypeStruct((NC, NS, RPT, H), jnp.float32),
        mesh=_sc_mesh(),
        compiler_params=_SC_PARAMS,
        scratch_types=[
            pltpu.VMEM((CH, CK), jnp.int32),
            pltpu.VMEM((CH, CK), jnp.int32),
            pltpu.VMEM((NB, CK, H), jnp.float32),
            pltpu.VMEM_SHARED((NP, H), jnp.float32),
            pltpu.SemaphoreType.DMA,
        ] + [pltpu.SemaphoreType.DMA] * (2 * NB),
    )
    return f(g4, h, zerosH)


def _enc1_body(x_ref, w_ref, dinv_ref, o_ref):
    h = jnp.dot(x_ref[...], w_ref[...], preferred_element_type=jnp.float32)
    o_ref[...] = h * dinv_ref[...]


def _enc1(X, W1, dinv):
    return pl.pallas_call(
        _enc1_body,
        grid=(N // BN,),
        in_specs=[
            pl.BlockSpec((BN, D), lambda i: (i, 0)),
            pl.BlockSpec((D, H), lambda i: (0, 0)),
            pl.BlockSpec((BN, H), lambda i: (i, 0)),
        ],
        out_specs=pl.BlockSpec((BN, H), lambda i: (i, 0)),
        out_shape=jax.ShapeDtypeStruct((N, H), jnp.float32),
    )(X, W1, dinv)


def _enc2_body(p0_ref, p1_ref, hs_ref, dinv_ref, b1_ref, o_ref):
    dinv = dinv_ref[...]
    agg = (p0_ref[...] + p1_ref[...] + hs_ref[...]) * dinv + b1_ref[...]
    o_ref[...] = jnp.maximum(agg, 0.0) * dinv


def _enc2(p0, p1, h1s, dinv, b1):
    bspec = pl.BlockSpec((BN, H), lambda i: (i, 0))
    return pl.pallas_call(
        _enc2_body,
        grid=(N // BN,),
        in_specs=[
            bspec, bspec, bspec,
            bspec,
            pl.BlockSpec((1, H), lambda i: (0, 0)),
        ],
        out_specs=bspec,
        out_shape=jax.ShapeDtypeStruct((N, H), jnp.float32),
    )(p0, p1, h1s, dinv, b1)


def _enc3_body(q0_ref, q1_ref, x1s_ref, dinv_ref, wmu_ref, bmu_ref,
               ws_ref, bs_ref, eps_ref, mus_o, ls_o, z_o):
    p = (q0_ref[...] + q1_ref[...] + x1s_ref[...]) * dinv_ref[...]
    mus = jnp.dot(p, wmu_ref[...], preferred_element_type=jnp.float32) + bmu_ref[...]
    ls = jnp.dot(p, ws_ref[...], preferred_element_type=jnp.float32) + bs_ref[...]
    mus_o[...] = mus
    ls_o[...] = ls
    z_o[...] = mus + jnp.exp(0.5 * ls) * eps_ref[...]


def _enc3(q0, q1, x1s, dinv, Wmu, bmu, Ws, bs, eps):
    bspecH = pl.BlockSpec((BN, H), lambda i: (i, 0))
    return pl.pallas_call(
        _enc3_body,
        grid=(N // BN,),
        in_specs=[
            bspecH, bspecH, bspecH,
            bspecH,
            pl.BlockSpec((H, L), lambda i: (0, 0)),
            pl.BlockSpec((1, L), lambda i: (0, 0)),
            pl.BlockSpec((H, 1), lambda i: (0, 0)),
            pl.BlockSpec((1, 1), lambda i: (0, 0)),
            pl.BlockSpec((BN, L), lambda i: (i, 0)),
        ],
        out_specs=[
            pl.BlockSpec((BN, L), lambda i: (i, 0)),
            pl.BlockSpec((BN, 1), lambda i: (i, 0)),
            pl.BlockSpec((BN, L), lambda i: (i, 0)),
        ],
        out_shape=[
            jax.ShapeDtypeStruct((N, L), jnp.float32),
            jax.ShapeDtypeStruct((N, 1), jnp.float32),
            jax.ShapeDtypeStruct((N, L), jnp.float32),
        ],
    )(q0, q1, x1s, dinv, Wmu, bmu, Ws, bs, eps)


def _zzt_body(a_ref, bt_ref, o_ref):
    o_ref[...] = jnp.dot(a_ref[...], bt_ref[...],
                         preferred_element_type=jnp.float32)


def _zzt(Z, ZT):
    bm = 200
    return pl.pallas_call(
        _zzt_body,
        grid=(N // bm,),
        in_specs=[
            pl.BlockSpec((bm, L), lambda i: (i, 0)),
            pl.BlockSpec((L, N), lambda i: (0, 0)),
        ],
        out_specs=pl.BlockSpec((bm, N), lambda i: (i, 0)),
        out_shape=jax.ShapeDtypeStruct((N, N), jnp.float32),
    )(Z, ZT)


def kernel(X, graph, W1, b1, Wmu, bmu, Ws, bs):
    graph = graph.astype(jnp.int32)
    g4 = graph.reshape(2, NW, CH, CK)
    ones_k = jnp.ones((CK,), jnp.float32)
    zeros1 = jnp.zeros((RPT,), jnp.float32)
    zerosH = jnp.zeros((RPT, H), jnp.float32)

    degp = _deg_call(g4, ones_k, zeros1).reshape(NC, NP)
    deg = degp[0, :N] + degp[1, :N] + 1.0
    dinvH = jnp.broadcast_to(lax.rsqrt(deg)[:, None], (N, H))

    h1s = _enc1(X, W1, dinvH)
    p = _agg_call(g4, h1s, zerosH).reshape(NC, NP, H)[:, :N]
    x1s = _enc2(p[0], p[1], h1s, dinvH, b1.reshape(1, H))
    q = _agg_call(g4, x1s, zerosH).reshape(NC, NP, H)[:, :N]

    eps = jax.random.normal(jax.random.key(1), (N, L), jnp.float32)
    mus, logsigma2s, Z = _enc3(q[0], q[1], x1s, dinvH, Wmu,
                               bmu.reshape(1, L), Ws, bs.reshape(1, 1), eps)
    ZZt = _zzt(Z, Z.T)
    return (ZZt, mus, logsigma2s)
